# baseline (speedup 1.0000x reference)
import functools

import jax
import jax.numpy as jnp
from jax import lax
from jax.experimental import pallas as pl
from jax.experimental.pallas import tpu as pltpu
from jax.experimental.pallas import tpu_sc as plsc

NE = 8
DM = 2048
DFF = 1024
NTOK = 2048
BLK = 256
NSLOT = NTOK * 2 + NE * BLK
NBLK = NSLOT // BLK
TBLK = 256

SC_NC = 2
SC_NS = 16
SC_NW = SC_NC * SC_NS
G_CH = 16


def _router_body(x_ref, g_ref, b_ref, w_ref):
    x = x_ref[...]
    logits = lax.dot_general(x, g_ref[...], (((1,), (1,)), ((), ())),
                             preferred_element_type=jnp.float32)
    rw = jax.nn.sigmoid(logits)
    scores = rw + b_ref[...]
    ii = lax.broadcasted_iota(jnp.int32, scores.shape, 1)
    m1 = jnp.max(scores, axis=1, keepdims=True)
    i1 = jnp.min(jnp.where(scores == m1, ii, NE), axis=1, keepdims=True)
    s2 = jnp.where(ii == i1, -jnp.inf, scores)
    m2 = jnp.max(s2, axis=1, keepdims=True)
    i2 = jnp.min(jnp.where(s2 == m2, ii, NE), axis=1, keepdims=True)
    rw1 = jnp.sum(jnp.where(ii == i1, rw, 0.0), axis=1, keepdims=True)
    rw2 = jnp.sum(jnp.where(ii == i2, rw, 0.0), axis=1, keepdims=True)
    tot = rw1 + rw2
    w_ref[...] = (jnp.where(ii == i1, rw1 / tot, 0.0)
                  + jnp.where(ii == i2, rw2 / tot, 0.0))


def _router(x, gate_w, bias):
    return pl.pallas_call(
        _router_body,
        grid=(NTOK // TBLK,),
        in_specs=[
            pl.BlockSpec((TBLK, DM), lambda b: (b, 0)),
            pl.BlockSpec((NE, DM), lambda b: (0, 0)),
            pl.BlockSpec((1, NE), lambda b: (0, 0)),
        ],
        out_specs=pl.BlockSpec((TBLK, NE), lambda b: (b, 0)),
        out_shape=jax.ShapeDtypeStruct((NTOK, NE), jnp.float32),
    )(x, gate_w, bias.reshape(1, NE))


def _dispatch_plan(w_dense):
    mask = w_dense > 0.0
    mi = mask.astype(jnp.int32)
    counts = jnp.sum(mi, axis=0)
    padded = ((counts + BLK - 1) // BLK) * BLK
    pstart = jnp.concatenate(
        [jnp.zeros((1,), jnp.int32), jnp.cumsum(padded)[:-1].astype(jnp.int32)])
    prefix = jnp.cumsum(mi, axis=0) - mi
    slot = pstart[None, :] + prefix
    ts0 = jnp.min(jnp.where(mask, slot, NSLOT), axis=1).astype(jnp.int32)
    ts1 = jnp.max(jnp.where(mask, slot, -1), axis=1).astype(jnp.int32)
    w0 = jnp.sum(jnp.where(mask & (slot == ts0[:, None]), w_dense, 0.0), axis=1)
    w1 = jnp.sum(jnp.where(mask & (slot == ts1[:, None]), w_dense, 0.0), axis=1)
    bend = jnp.cumsum(padded // BLK)
    bidx = jnp.arange(NBLK, dtype=jnp.int32)
    eb = jnp.minimum(
        jnp.sum((bidx[:, None] >= bend[None, :]).astype(jnp.int32), axis=1),
        NE - 1).astype(jnp.int32)
    ebx = jnp.concatenate([eb, bend[-1:].astype(jnp.int32)])
    return ts0, ts1, w0, w1, ebx


C_D = 8


def _sc_dispatch(x, t0_3d, t1_3d):
    tok_pw = NTOK // SC_NW
    nchunk = tok_pw // C_D
    nb = 3
    mesh = plsc.VectorSubcoreMesh(core_axis_name="c", subcore_axis_name="s")

    @functools.partial(
        pl.kernel, mesh=mesh,
        out_type=jax.ShapeDtypeStruct((NSLOT, DM), jnp.float32),
        scratch_types=[
            pltpu.VMEM((nchunk, C_D), jnp.int32),
            pltpu.VMEM((nchunk, C_D), jnp.int32),
            pltpu.VMEM((C_D, DM), jnp.float32),
            pltpu.VMEM((C_D, DM), jnp.float32),
            pltpu.VMEM((C_D, DM), jnp.float32),
            pltpu.SemaphoreType.DMA,
            pltpu.SemaphoreType.DMA,
            pltpu.SemaphoreType.DMA,
            pltpu.SemaphoreType.DMA,
            pltpu.SemaphoreType.DMA,
            pltpu.SemaphoreType.DMA,
            pltpu.SemaphoreType.DMA,
            pltpu.SemaphoreType.DMA,
            pltpu.SemaphoreType.DMA,
        ],
    )
    def k(x_hbm, t0_hbm, t1_hbm, xs_hbm, i0_v, i1_v, b0, b1, b2,
          r0, r1, r2, s00, s01, s02, s10, s11, s12):
        bufs = (b0, b1, b2)
        rsems = (r0, r1, r2)
        s0sems = (s00, s01, s02)
        s1sems = (s10, s11, s12)
        wid = lax.axis_index("s") * SC_NC + lax.axis_index("c")
        base = wid * tok_pw
        pltpu.sync_copy(t0_hbm.at[wid], i0_v)
        pltpu.sync_copy(t1_hbm.at[wid], i1_v)
        rh = [None] * nchunk
        s0h = [None] * nchunk
        s1h = [None] * nchunk
        for i in range(nchunk):
            b = i % nb
            if i >= nb:
                s0h[i - nb].wait()
                s1h[i - nb].wait()
            rh[i] = pltpu.async_copy(
                x_hbm.at[pl.ds(base + i * C_D, C_D)], bufs[b], rsems[b])
            if i >= 1:
                pb = (i - 1) % nb
                rh[i - 1].wait()
                s0h[i - 1] = pltpu.async_copy(
                    bufs[pb], xs_hbm.at[i0_v.at[i - 1]], s0sems[pb])
                s1h[i - 1] = pltpu.async_copy(
                    bufs[pb], xs_hbm.at[i1_v.at[i - 1]], s1sems[pb])
        last = nchunk - 1
        rh[last].wait()
        s0h[last] = pltpu.async_copy(
            bufs[last % nb], xs_hbm.at[i0_v.at[last]], s0sems[last % nb])
        s1h[last] = pltpu.async_copy(
            bufs[last % nb], xs_hbm.at[i1_v.at[last]], s1sems[last % nb])
        for i in range(nchunk - nb, nchunk):
            s0h[i].wait()
            s1h[i].wait()

    return k(x, t0_3d, t1_3d)


def _ffn_body(eb_ref, xs_ref, w1_ref, w2_ref, w3_ref, os_ref):
    @pl.when(pl.program_id(0) < eb_ref[NBLK])
    def _compute():
        xb = xs_ref[...].astype(jnp.bfloat16)
        a = lax.dot_general(xb, w1_ref[0].astype(jnp.bfloat16),
                            (((1,), (1,)), ((), ())),
                            preferred_element_type=jnp.float32)
        c = lax.dot_general(xb, w3_ref[0].astype(jnp.bfloat16),
                            (((1,), (1,)), ((), ())),
                            preferred_element_type=jnp.float32)
        h = a * jax.nn.sigmoid(a) * c
        o = lax.dot_general(h.astype(jnp.bfloat16),
                            w2_ref[0].astype(jnp.bfloat16),
                            (((1,), (1,)), ((), ())),
                            preferred_element_type=jnp.float32)
        os_ref[...] = o


def _ffn(xs, w1, w2, w3, eb):
    grid_spec = pltpu.PrefetchScalarGridSpec(
        num_scalar_prefetch=1,
        grid=(NBLK,),
        in_specs=[
            pl.BlockSpec((BLK, DM), lambda b, eb: (b, 0)),
            pl.BlockSpec((1, DFF, DM), lambda b, eb: (eb[b], 0, 0)),
            pl.BlockSpec((1, DM, DFF), lambda b, eb: (eb[b], 0, 0)),
            pl.BlockSpec((1, DFF, DM), lambda b, eb: (eb[b], 0, 0)),
        ],
        out_specs=pl.BlockSpec((BLK, DM), lambda b, eb: (b, 0)),
    )
    return pl.pallas_call(
        _ffn_body,
        grid_spec=grid_spec,
        out_shape=jax.ShapeDtypeStruct((NSLOT, DM), jnp.float32),
    )(eb, xs, w1, w2, w3)


C_CH = 8


def _sc_combine(os, ts0, ts1, w0r, w1r):
    tok_pw = NTOK // SC_NW
    nchunk = tok_pw // C_CH
    mesh = plsc.VectorSubcoreMesh(core_axis_name="c", subcore_axis_name="s")

    @functools.partial(
        pl.kernel, mesh=mesh,
        out_type=jax.ShapeDtypeStruct((NTOK, DM), jnp.float32),
        scratch_types=[
            pltpu.VMEM((tok_pw,), jnp.int32),
            pltpu.VMEM((tok_pw,), jnp.int32),
            pltpu.VMEM((tok_pw, 16), jnp.float32),
            pltpu.VMEM((tok_pw, 16), jnp.float32),
            pltpu.VMEM((C_CH, DM), jnp.float32),
            pltpu.VMEM((C_CH, DM), jnp.float32),
            pltpu.VMEM((C_CH, DM), jnp.float32),
            pltpu.VMEM((C_CH, DM), jnp.float32),
            pltpu.SemaphoreType.DMA,
            pltpu.SemaphoreType.DMA,
            pltpu.SemaphoreType.DMA,
            pltpu.SemaphoreType.DMA,
            pltpu.SemaphoreType.DMA,
            pltpu.SemaphoreType.DMA,
        ],
    )
    def k(os_hbm, t0_hbm, t1_hbm, w0_hbm, w1_hbm, out_hbm, i0_v, i1_v,
          w0_v, w1_v, a0, a1, c0, c1, sa0, sa1, sb0, sb1, so0, so1):
        abufs = (a0, a1)
        bbufs = (c0, c1)
        gasems = (sa0, sa1)
        gbsems = (sb0, sb1)
        wsems = (so0, so1)
        wid = lax.axis_index("s") * SC_NC + lax.axis_index("c")
        base = wid * tok_pw
        pltpu.sync_copy(t0_hbm.at[pl.ds(base, tok_pw)], i0_v)
        pltpu.sync_copy(t1_hbm.at[pl.ds(base, tok_pw)], i1_v)
        pltpu.sync_copy(w0_hbm.at[pl.ds(base, tok_pw)], w0_v)
        pltpu.sync_copy(w1_hbm.at[pl.ds(base, tok_pw)], w1_v)

        def do_adds(b, ci):
            w0s = [w0_v[ci * C_CH + r, :] for r in range(C_CH)]
            w1s = [w1_v[ci * C_CH + r, :] for r in range(C_CH)]

            def add_col(j, carry):
                for r in range(C_CH):
                    abufs[b][r, pl.ds(j * 16, 16)] = (
                        abufs[b][r, pl.ds(j * 16, 16)] * w0s[r]
                        + bbufs[b][r, pl.ds(j * 16, 16)] * w1s[r])
                return carry
            lax.fori_loop(0, DM // 16, add_col, 0)

        gha = [None] * nchunk
        ghb = [None] * nchunk
        wh = [None] * nchunk
        for i in range(nchunk):
            b = i % 2
            if i >= 2:
                wh[i - 2].wait()
            gha[i] = pltpu.async_copy(
                os_hbm.at[i0_v.at[pl.ds(i * C_CH, C_CH)]], abufs[b],
                gasems[b])
            ghb[i] = pltpu.async_copy(
                os_hbm.at[i1_v.at[pl.ds(i * C_CH, C_CH)]], bbufs[b],
                gbsems[b])
            if i >= 1:
                pb = (i - 1) % 2
                gha[i - 1].wait()
                ghb[i - 1].wait()
                do_adds(pb, i - 1)
                wh[i - 1] = pltpu.async_copy(
                    abufs[pb], out_hbm.at[pl.ds(base + (i - 1) * C_CH, C_CH)],
                    wsems[pb])
        last = nchunk - 1
        gha[last].wait()
        ghb[last].wait()
        do_adds(last % 2, last)
        wh[last] = pltpu.async_copy(
            abufs[last % 2], out_hbm.at[pl.ds(base + last * C_CH, C_CH)],
            wsems[last % 2])
        wh[nchunk - 2].wait()
        wh[last].wait()

    return k(os, ts0, ts1, w0r, w1r)


def kernel(hidden_states, gate_w, e_score_correction_bias, w1, w2, w3):
    bsz, seq, dm = hidden_states.shape
    x = hidden_states.reshape(-1, dm)
    w_dense = _router(x, gate_w, e_score_correction_bias)
    ts0, ts1, w0, w1r, ebx = _dispatch_plan(w_dense)
    nch = NTOK // SC_NW // C_D
    xs = _sc_dispatch(x, ts0.reshape(SC_NW, nch, C_D),
                      ts1.reshape(SC_NW, nch, C_D))
    os = _ffn(xs, w1, w2, w3, ebx)
    final = _sc_combine(os, ts0, ts1,
                        jnp.broadcast_to(w0[:, None], (NTOK, 16)),
                        jnp.broadcast_to(w1r[:, None], (NTOK, 16)))
    return final.reshape(bsz, seq, dm)


# speedup vs baseline: 1.0016x; 1.0016x over previous
import functools

import jax
import jax.numpy as jnp
from jax import lax
from jax.experimental import pallas as pl
from jax.experimental.pallas import tpu as pltpu
from jax.experimental.pallas import tpu_sc as plsc

NE = 8
DM = 2048
DFF = 1024
NTOK = 2048
BLK = 256
NSLOT = NTOK * 2 + NE * BLK
NBLK = NSLOT // BLK
TBLK = 256

SC_NC = 2
SC_NS = 16
SC_NW = SC_NC * SC_NS
G_CH = 16


def _router_body(x_ref, g_ref, b_ref, w_ref):
    x = x_ref[...]
    logits = lax.dot_general(x, g_ref[...], (((1,), (1,)), ((), ())),
                             preferred_element_type=jnp.float32)
    rw = jax.nn.sigmoid(logits)
    scores = rw + b_ref[...]
    ii = lax.broadcasted_iota(jnp.int32, scores.shape, 1)
    m1 = jnp.max(scores, axis=1, keepdims=True)
    i1 = jnp.min(jnp.where(scores == m1, ii, NE), axis=1, keepdims=True)
    s2 = jnp.where(ii == i1, -jnp.inf, scores)
    m2 = jnp.max(s2, axis=1, keepdims=True)
    i2 = jnp.min(jnp.where(s2 == m2, ii, NE), axis=1, keepdims=True)
    rw1 = jnp.sum(jnp.where(ii == i1, rw, 0.0), axis=1, keepdims=True)
    rw2 = jnp.sum(jnp.where(ii == i2, rw, 0.0), axis=1, keepdims=True)
    tot = rw1 + rw2
    w_ref[...] = (jnp.where(ii == i1, rw1 / tot, 0.0)
                  + jnp.where(ii == i2, rw2 / tot, 0.0))


def _router(x, gate_w, bias):
    return pl.pallas_call(
        _router_body,
        grid=(NTOK // TBLK,),
        in_specs=[
            pl.BlockSpec((TBLK, DM), lambda b: (b, 0)),
            pl.BlockSpec((NE, DM), lambda b: (0, 0)),
            pl.BlockSpec((1, NE), lambda b: (0, 0)),
        ],
        out_specs=pl.BlockSpec((TBLK, NE), lambda b: (b, 0)),
        out_shape=jax.ShapeDtypeStruct((NTOK, NE), jnp.float32),
    )(x, gate_w, bias.reshape(1, NE))


def _dispatch_plan(w_dense):
    mask = w_dense > 0.0
    mi = mask.astype(jnp.int32)
    counts = jnp.sum(mi, axis=0)
    padded = ((counts + BLK - 1) // BLK) * BLK
    pstart = jnp.concatenate(
        [jnp.zeros((1,), jnp.int32), jnp.cumsum(padded)[:-1].astype(jnp.int32)])
    prefix = jnp.cumsum(mi, axis=0) - mi
    slot = pstart[None, :] + prefix
    ts0 = jnp.min(jnp.where(mask, slot, NSLOT), axis=1).astype(jnp.int32)
    ts1 = jnp.max(jnp.where(mask, slot, -1), axis=1).astype(jnp.int32)
    w0 = jnp.sum(jnp.where(mask & (slot == ts0[:, None]), w_dense, 0.0), axis=1)
    w1 = jnp.sum(jnp.where(mask & (slot == ts1[:, None]), w_dense, 0.0), axis=1)
    bend = jnp.cumsum(padded // BLK)
    bidx = jnp.arange(NBLK, dtype=jnp.int32)
    eb = jnp.minimum(
        jnp.sum((bidx[:, None] >= bend[None, :]).astype(jnp.int32), axis=1),
        NE - 1).astype(jnp.int32)
    ebx = jnp.concatenate([eb, bend[-1:].astype(jnp.int32)])
    return ts0, ts1, w0, w1, ebx


C_D = 8
WREP = 128


def _sc_dispatch(x, t0_3d, t1_3d, ts0, ts1, w0r, w1r):
    tok_pw = NTOK // SC_NW
    nchunk = tok_pw // C_D
    nb = 3
    mesh = plsc.VectorSubcoreMesh(core_axis_name="c", subcore_axis_name="s")

    @functools.partial(
        pl.kernel, mesh=mesh,
        out_type=(jax.ShapeDtypeStruct((NSLOT, DM), jnp.float32),
                  jax.ShapeDtypeStruct((NSLOT, WREP), jnp.float32)),
        scratch_types=[
            pltpu.VMEM((nchunk, C_D), jnp.int32),
            pltpu.VMEM((nchunk, C_D), jnp.int32),
            pltpu.VMEM((tok_pw,), jnp.int32),
            pltpu.VMEM((tok_pw,), jnp.int32),
            pltpu.VMEM((tok_pw, WREP), jnp.float32),
            pltpu.VMEM((tok_pw, WREP), jnp.float32),
            pltpu.VMEM((C_D, DM), jnp.float32),
            pltpu.VMEM((C_D, DM), jnp.float32),
            pltpu.VMEM((C_D, DM), jnp.float32),
            pltpu.SemaphoreType.DMA,
            pltpu.SemaphoreType.DMA,
            pltpu.SemaphoreType.DMA,
            pltpu.SemaphoreType.DMA,
            pltpu.SemaphoreType.DMA,
            pltpu.SemaphoreType.DMA,
            pltpu.SemaphoreType.DMA,
            pltpu.SemaphoreType.DMA,
            pltpu.SemaphoreType.DMA,
            pltpu.SemaphoreType.DMA,
        ],
    )
    def k(x_hbm, t0_hbm, t1_hbm, tf0_hbm, tf1_hbm, w0_hbm, w1_hbm,
          xs_hbm, ws_hbm, i0_v, i1_v, if0_v, if1_v, w0_v, w1_v, b0, b1, b2,
          r0, r1, r2, s00, s01, s02, s10, s11, s12, wsem):
        bufs = (b0, b1, b2)
        rsems = (r0, r1, r2)
        s0sems = (s00, s01, s02)
        s1sems = (s10, s11, s12)
        wid = lax.axis_index("s") * SC_NC + lax.axis_index("c")
        base = wid * tok_pw
        pltpu.sync_copy(t0_hbm.at[wid], i0_v)
        pltpu.sync_copy(t1_hbm.at[wid], i1_v)
        pltpu.sync_copy(tf0_hbm.at[pl.ds(base, tok_pw)], if0_v)
        pltpu.sync_copy(tf1_hbm.at[pl.ds(base, tok_pw)], if1_v)
        pltpu.sync_copy(w0_hbm.at[pl.ds(base, tok_pw)], w0_v)
        pltpu.sync_copy(w1_hbm.at[pl.ds(base, tok_pw)], w1_v)
        wh0 = pltpu.async_copy(w0_v, ws_hbm.at[if0_v], wsem)
        wh1 = pltpu.async_copy(w1_v, ws_hbm.at[if1_v], wsem)
        rh = [None] * nchunk
        s0h = [None] * nchunk
        s1h = [None] * nchunk
        for i in range(nchunk):
            b = i % nb
            if i >= nb:
                s0h[i - nb].wait()
                s1h[i - nb].wait()
            rh[i] = pltpu.async_copy(
                x_hbm.at[pl.ds(base + i * C_D, C_D)], bufs[b], rsems[b])
            if i >= 1:
                pb = (i - 1) % nb
                rh[i - 1].wait()
                s0h[i - 1] = pltpu.async_copy(
                    bufs[pb], xs_hbm.at[i0_v.at[i - 1]], s0sems[pb])
                s1h[i - 1] = pltpu.async_copy(
                    bufs[pb], xs_hbm.at[i1_v.at[i - 1]], s1sems[pb])
        last = nchunk - 1
        rh[last].wait()
        s0h[last] = pltpu.async_copy(
            bufs[last % nb], xs_hbm.at[i0_v.at[last]], s0sems[last % nb])
        s1h[last] = pltpu.async_copy(
            bufs[last % nb], xs_hbm.at[i1_v.at[last]], s1sems[last % nb])
        for i in range(nchunk - nb, nchunk):
            s0h[i].wait()
            s1h[i].wait()
        wh0.wait()
        wh1.wait()

    return k(x, t0_3d, t1_3d, ts0, ts1, w0r, w1r)


def _ffn_body(eb_ref, xs_ref, ws_ref, w1_ref, w2_ref, w3_ref, os_ref):
    @pl.when(pl.program_id(0) < eb_ref[NBLK])
    def _compute():
        xb = xs_ref[...]
        a = lax.dot_general(xb, w1_ref[0], (((1,), (1,)), ((), ())),
                            preferred_element_type=jnp.float32)
        c = lax.dot_general(xb, w3_ref[0], (((1,), (1,)), ((), ())),
                            preferred_element_type=jnp.float32)
        h = a * jax.nn.sigmoid(a) * c
        o = lax.dot_general(h, w2_ref[0], (((1,), (1,)), ((), ())),
                            preferred_element_type=jnp.float32)
        os_ref[...] = o * ws_ref[:, :1]


def _ffn(xs, ws_rep, w1, w2, w3, eb):
    grid_spec = pltpu.PrefetchScalarGridSpec(
        num_scalar_prefetch=1,
        grid=(NBLK,),
        in_specs=[
            pl.BlockSpec((BLK, DM), lambda b, eb: (b, 0)),
            pl.BlockSpec((BLK, WREP), lambda b, eb: (b, 0)),
            pl.BlockSpec((1, DFF, DM), lambda b, eb: (eb[b], 0, 0)),
            pl.BlockSpec((1, DM, DFF), lambda b, eb: (eb[b], 0, 0)),
            pl.BlockSpec((1, DFF, DM), lambda b, eb: (eb[b], 0, 0)),
        ],
        out_specs=pl.BlockSpec((BLK, DM), lambda b, eb: (b, 0)),
    )
    return pl.pallas_call(
        _ffn_body,
        grid_spec=grid_spec,
        out_shape=jax.ShapeDtypeStruct((NSLOT, DM), jnp.float32),
    )(eb, xs, ws_rep, w1, w2, w3)


C_CH = 8


def _sc_combine(os, ts0, ts1):
    tok_pw = NTOK // SC_NW
    nchunk = tok_pw // C_CH
    nb = 3
    mesh = plsc.VectorSubcoreMesh(core_axis_name="c", subcore_axis_name="s")

    @functools.partial(
        pl.kernel, mesh=mesh,
        out_type=jax.ShapeDtypeStruct((NTOK, DM), jnp.float32),
        scratch_types=[
            pltpu.VMEM((tok_pw,), jnp.int32),
            pltpu.VMEM((tok_pw,), jnp.int32),
            pltpu.VMEM((C_CH, DM), jnp.float32),
            pltpu.VMEM((C_CH, DM), jnp.float32),
            pltpu.VMEM((C_CH, DM), jnp.float32),
            pltpu.VMEM((C_CH, DM), jnp.float32),
            pltpu.VMEM((C_CH, DM), jnp.float32),
            pltpu.VMEM((C_CH, DM), jnp.float32),
            pltpu.SemaphoreType.DMA,
            pltpu.SemaphoreType.DMA,
            pltpu.SemaphoreType.DMA,
            pltpu.SemaphoreType.DMA,
            pltpu.SemaphoreType.DMA,
            pltpu.SemaphoreType.DMA,
            pltpu.SemaphoreType.DMA,
            pltpu.SemaphoreType.DMA,
            pltpu.SemaphoreType.DMA,
        ],
    )
    def k(os_hbm, t0_hbm, t1_hbm, out_hbm, i0_v, i1_v, a0, a1, a2,
          c0, c1, c2, sa0, sa1, sa2, sb0, sb1, sb2, so0, so1, so2):
        abufs = (a0, a1, a2)
        bbufs = (c0, c1, c2)
        gasems = (sa0, sa1, sa2)
        gbsems = (sb0, sb1, sb2)
        wsems = (so0, so1, so2)
        wid = lax.axis_index("s") * SC_NC + lax.axis_index("c")
        base = wid * tok_pw
        pltpu.sync_copy(t0_hbm.at[pl.ds(base, tok_pw)], i0_v)
        pltpu.sync_copy(t1_hbm.at[pl.ds(base, tok_pw)], i1_v)

        def do_adds(b):
            def add_col(j, carry):
                for r in range(C_CH):
                    abufs[b][r, pl.ds(j * 16, 16)] = (
                        abufs[b][r, pl.ds(j * 16, 16)]
                        + bbufs[b][r, pl.ds(j * 16, 16)])
                return carry
            lax.fori_loop(0, DM // 16, add_col, 0)

        gha = [None] * nchunk
        ghb = [None] * nchunk
        wh = [None] * nchunk
        for i in range(nchunk):
            b = i % nb
            if i >= nb:
                wh[i - nb].wait()
            gha[i] = pltpu.async_copy(
                os_hbm.at[i0_v.at[pl.ds(i * C_CH, C_CH)]], abufs[b],
                gasems[b])
            ghb[i] = pltpu.async_copy(
                os_hbm.at[i1_v.at[pl.ds(i * C_CH, C_CH)]], bbufs[b],
                gbsems[b])
            if i >= 1:
                pb = (i - 1) % nb
                gha[i - 1].wait()
                ghb[i - 1].wait()
                do_adds(pb)
                wh[i - 1] = pltpu.async_copy(
                    abufs[pb], out_hbm.at[pl.ds(base + (i - 1) * C_CH, C_CH)],
                    wsems[pb])
        last = nchunk - 1
        gha[last].wait()
        ghb[last].wait()
        do_adds(last % nb)
        wh[last] = pltpu.async_copy(
            abufs[last % nb], out_hbm.at[pl.ds(base + last * C_CH, C_CH)],
            wsems[last % nb])
        for i in range(nchunk - nb, nchunk):
            wh[i].wait()

    return k(os, ts0, ts1)


def kernel(hidden_states, gate_w, e_score_correction_bias, w1, w2, w3):
    bsz, seq, dm = hidden_states.shape
    x = hidden_states.reshape(-1, dm)
    w_dense = _router(x, gate_w, e_score_correction_bias)
    ts0, ts1, w0, w1r, ebx = _dispatch_plan(w_dense)
    nch = NTOK // SC_NW // C_D
    xs, ws_rep = _sc_dispatch(
        x, ts0.reshape(SC_NW, nch, C_D), ts1.reshape(SC_NW, nch, C_D),
        ts0, ts1,
        jnp.broadcast_to(w0[:, None], (NTOK, WREP)),
        jnp.broadcast_to(w1r[:, None], (NTOK, WREP)))
    os = _ffn(xs, ws_rep, w1, w2, w3, ebx)
    final = _sc_combine(os, ts0, ts1)
    return final.reshape(bsz, seq, dm)


# speedup vs baseline: 1.0218x; 1.0202x over previous
import functools

import jax
import jax.numpy as jnp
from jax import lax
from jax.experimental import pallas as pl
from jax.experimental.pallas import tpu as pltpu
from jax.experimental.pallas import tpu_sc as plsc

NE = 8
DM = 2048
DFF = 1024
NTOK = 2048
BLK = 256
NSLOT = NTOK * 2 + NE * BLK
NBLK = NSLOT // BLK
TBLK = 256

SC_NC = 2
SC_NS = 16
SC_NW = SC_NC * SC_NS
G_CH = 16


def _router_body(x_ref, g_ref, b_ref, w_ref, wr0_ref, wr1_ref):
    x = x_ref[...]
    logits = lax.dot_general(x, g_ref[...], (((1,), (1,)), ((), ())),
                             preferred_element_type=jnp.float32)
    rw = jax.nn.sigmoid(logits)
    scores = rw + b_ref[...]
    ii = lax.broadcasted_iota(jnp.int32, scores.shape, 1)
    m1 = jnp.max(scores, axis=1, keepdims=True)
    i1 = jnp.min(jnp.where(scores == m1, ii, NE), axis=1, keepdims=True)
    s2 = jnp.where(ii == i1, -jnp.inf, scores)
    m2 = jnp.max(s2, axis=1, keepdims=True)
    i2 = jnp.min(jnp.where(s2 == m2, ii, NE), axis=1, keepdims=True)
    rw1 = jnp.sum(jnp.where(ii == i1, rw, 0.0), axis=1, keepdims=True)
    rw2 = jnp.sum(jnp.where(ii == i2, rw, 0.0), axis=1, keepdims=True)
    tot = rw1 + rw2
    w_ref[...] = (jnp.where(ii == i1, rw1 / tot, 0.0)
                  + jnp.where(ii == i2, rw2 / tot, 0.0))
    wlo = jnp.where(i1 < i2, rw1, rw2) / tot
    whi = jnp.where(i1 < i2, rw2, rw1) / tot
    wr0_ref[...] = lax.broadcast_in_dim(wlo, (TBLK, WREP), (0, 1))
    wr1_ref[...] = lax.broadcast_in_dim(whi, (TBLK, WREP), (0, 1))


def _router(x, gate_w, bias):
    return pl.pallas_call(
        _router_body,
        grid=(NTOK // TBLK,),
        in_specs=[
            pl.BlockSpec((TBLK, DM), lambda b: (b, 0)),
            pl.BlockSpec((NE, DM), lambda b: (0, 0)),
            pl.BlockSpec((1, NE), lambda b: (0, 0)),
        ],
        out_specs=[
            pl.BlockSpec((TBLK, NE), lambda b: (b, 0)),
            pl.BlockSpec((TBLK, WREP), lambda b: (b, 0)),
            pl.BlockSpec((TBLK, WREP), lambda b: (b, 0)),
        ],
        out_shape=[
            jax.ShapeDtypeStruct((NTOK, NE), jnp.float32),
            jax.ShapeDtypeStruct((NTOK, WREP), jnp.float32),
            jax.ShapeDtypeStruct((NTOK, WREP), jnp.float32),
        ],
    )(x, gate_w, bias.reshape(1, NE))


def _dispatch_plan(w_dense):
    mask = w_dense > 0.0
    mi = mask.astype(jnp.int32)
    counts = jnp.sum(mi, axis=0)
    padded = ((counts + BLK - 1) // BLK) * BLK
    pstart = jnp.concatenate(
        [jnp.zeros((1,), jnp.int32), jnp.cumsum(padded)[:-1].astype(jnp.int32)])
    prefix = jnp.cumsum(mi, axis=0) - mi
    slot = pstart[None, :] + prefix
    ts0 = jnp.min(jnp.where(mask, slot, NSLOT), axis=1).astype(jnp.int32)
    ts1 = jnp.max(jnp.where(mask, slot, -1), axis=1).astype(jnp.int32)
    bend = jnp.cumsum(padded // BLK)
    bidx = jnp.arange(NBLK, dtype=jnp.int32)
    eb = jnp.minimum(
        jnp.sum((bidx[:, None] >= bend[None, :]).astype(jnp.int32), axis=1),
        NE - 1).astype(jnp.int32)
    ebx = jnp.concatenate([eb, bend[-1:].astype(jnp.int32)])
    return ts0, ts1, ebx


C_D = 8
WREP = 128


def _sc_dispatch(x, t0_3d, t1_3d, ts0, ts1, w0r, w1r):
    tok_pw = NTOK // SC_NW
    nchunk = tok_pw // C_D
    nb = 3
    mesh = plsc.VectorSubcoreMesh(core_axis_name="c", subcore_axis_name="s")

    @functools.partial(
        pl.kernel, mesh=mesh,
        out_type=(jax.ShapeDtypeStruct((NSLOT, DM), jnp.float32),
                  jax.ShapeDtypeStruct((NSLOT, WREP), jnp.float32)),
        scratch_types=[
            pltpu.VMEM((nchunk, C_D), jnp.int32),
            pltpu.VMEM((nchunk, C_D), jnp.int32),
            pltpu.VMEM((tok_pw,), jnp.int32),
            pltpu.VMEM((tok_pw,), jnp.int32),
            pltpu.VMEM((tok_pw, WREP), jnp.float32),
            pltpu.VMEM((tok_pw, WREP), jnp.float32),
            pltpu.VMEM((C_D, DM), jnp.float32),
            pltpu.VMEM((C_D, DM), jnp.float32),
            pltpu.VMEM((C_D, DM), jnp.float32),
            pltpu.SemaphoreType.DMA,
            pltpu.SemaphoreType.DMA,
            pltpu.SemaphoreType.DMA,
            pltpu.SemaphoreType.DMA,
            pltpu.SemaphoreType.DMA,
            pltpu.SemaphoreType.DMA,
            pltpu.SemaphoreType.DMA,
            pltpu.SemaphoreType.DMA,
            pltpu.SemaphoreType.DMA,
            pltpu.SemaphoreType.DMA,
        ],
    )
    def k(x_hbm, t0_hbm, t1_hbm, tf0_hbm, tf1_hbm, w0_hbm, w1_hbm,
          xs_hbm, ws_hbm, i0_v, i1_v, if0_v, if1_v, w0_v, w1_v, b0, b1, b2,
          r0, r1, r2, s00, s01, s02, s10, s11, s12, wsem):
        bufs = (b0, b1, b2)
        rsems = (r0, r1, r2)
        s0sems = (s00, s01, s02)
        s1sems = (s10, s11, s12)
        wid = lax.axis_index("s") * SC_NC + lax.axis_index("c")
        base = wid * tok_pw
        pltpu.sync_copy(t0_hbm.at[wid], i0_v)
        pltpu.sync_copy(t1_hbm.at[wid], i1_v)
        pltpu.sync_copy(tf0_hbm.at[pl.ds(base, tok_pw)], if0_v)
        pltpu.sync_copy(tf1_hbm.at[pl.ds(base, tok_pw)], if1_v)
        pltpu.sync_copy(w0_hbm.at[pl.ds(base, tok_pw)], w0_v)
        pltpu.sync_copy(w1_hbm.at[pl.ds(base, tok_pw)], w1_v)
        wh0 = pltpu.async_copy(w0_v, ws_hbm.at[if0_v], wsem)
        wh1 = pltpu.async_copy(w1_v, ws_hbm.at[if1_v], wsem)
        rh = [None] * nchunk
        s0h = [None] * nchunk
        s1h = [None] * nchunk
        for i in range(nchunk):
            b = i % nb
            if i >= nb:
                s0h[i - nb].wait()
                s1h[i - nb].wait()
            rh[i] = pltpu.async_copy(
                x_hbm.at[pl.ds(base + i * C_D, C_D)], bufs[b], rsems[b])
            if i >= 1:
                pb = (i - 1) % nb
                rh[i - 1].wait()
                s0h[i - 1] = pltpu.async_copy(
                    bufs[pb], xs_hbm.at[i0_v.at[i - 1]], s0sems[pb])
                s1h[i - 1] = pltpu.async_copy(
                    bufs[pb], xs_hbm.at[i1_v.at[i - 1]], s1sems[pb])
        last = nchunk - 1
        rh[last].wait()
        s0h[last] = pltpu.async_copy(
            bufs[last % nb], xs_hbm.at[i0_v.at[last]], s0sems[last % nb])
        s1h[last] = pltpu.async_copy(
            bufs[last % nb], xs_hbm.at[i1_v.at[last]], s1sems[last % nb])
        for i in range(nchunk - nb, nchunk):
            s0h[i].wait()
            s1h[i].wait()
        wh0.wait()
        wh1.wait()

    return k(x, t0_3d, t1_3d, ts0, ts1, w0r, w1r)


def _ffn_body(eb_ref, xs_ref, ws_ref, w1_ref, w2_ref, w3_ref, os_ref):
    @pl.when(pl.program_id(0) < eb_ref[NBLK])
    def _compute():
        xb = xs_ref[...]
        a = lax.dot_general(xb, w1_ref[0], (((1,), (1,)), ((), ())),
                            preferred_element_type=jnp.float32)
        c = lax.dot_general(xb, w3_ref[0], (((1,), (1,)), ((), ())),
                            preferred_element_type=jnp.float32)
        h = a * jax.nn.sigmoid(a) * c
        o = lax.dot_general(h, w2_ref[0], (((1,), (1,)), ((), ())),
                            preferred_element_type=jnp.float32)
        os_ref[...] = o * ws_ref[:, :1]


def _ffn(xs, ws_rep, w1, w2, w3, eb):
    grid_spec = pltpu.PrefetchScalarGridSpec(
        num_scalar_prefetch=1,
        grid=(NBLK,),
        in_specs=[
            pl.BlockSpec((BLK, DM), lambda b, eb: (b, 0)),
            pl.BlockSpec((BLK, WREP), lambda b, eb: (b, 0)),
            pl.BlockSpec((1, DFF, DM), lambda b, eb: (eb[b], 0, 0)),
            pl.BlockSpec((1, DM, DFF), lambda b, eb: (eb[b], 0, 0)),
            pl.BlockSpec((1, DFF, DM), lambda b, eb: (eb[b], 0, 0)),
        ],
        out_specs=pl.BlockSpec((BLK, DM), lambda b, eb: (b, 0)),
    )
    return pl.pallas_call(
        _ffn_body,
        grid_spec=grid_spec,
        out_shape=jax.ShapeDtypeStruct((NSLOT, DM), jnp.float32),
    )(eb, xs, ws_rep, w1, w2, w3)


C_CH = 8


def _sc_combine(os, ts0, ts1):
    tok_pw = NTOK // SC_NW
    nchunk = tok_pw // C_CH
    nb = 3
    mesh = plsc.VectorSubcoreMesh(core_axis_name="c", subcore_axis_name="s")

    @functools.partial(
        pl.kernel, mesh=mesh,
        out_type=jax.ShapeDtypeStruct((NTOK, DM), jnp.float32),
        scratch_types=[
            pltpu.VMEM((tok_pw,), jnp.int32),
            pltpu.VMEM((tok_pw,), jnp.int32),
            pltpu.VMEM((C_CH, DM), jnp.float32),
            pltpu.VMEM((C_CH, DM), jnp.float32),
            pltpu.VMEM((C_CH, DM), jnp.float32),
            pltpu.VMEM((C_CH, DM), jnp.float32),
            pltpu.VMEM((C_CH, DM), jnp.float32),
            pltpu.VMEM((C_CH, DM), jnp.float32),
            pltpu.SemaphoreType.DMA,
            pltpu.SemaphoreType.DMA,
            pltpu.SemaphoreType.DMA,
            pltpu.SemaphoreType.DMA,
            pltpu.SemaphoreType.DMA,
            pltpu.SemaphoreType.DMA,
            pltpu.SemaphoreType.DMA,
            pltpu.SemaphoreType.DMA,
            pltpu.SemaphoreType.DMA,
        ],
    )
    def k(os_hbm, t0_hbm, t1_hbm, out_hbm, i0_v, i1_v, a0, a1, a2,
          c0, c1, c2, sa0, sa1, sa2, sb0, sb1, sb2, so0, so1, so2):
        abufs = (a0, a1, a2)
        bbufs = (c0, c1, c2)
        gasems = (sa0, sa1, sa2)
        gbsems = (sb0, sb1, sb2)
        wsems = (so0, so1, so2)
        wid = lax.axis_index("s") * SC_NC + lax.axis_index("c")
        base = wid * tok_pw
        pltpu.sync_copy(t0_hbm.at[pl.ds(base, tok_pw)], i0_v)
        pltpu.sync_copy(t1_hbm.at[pl.ds(base, tok_pw)], i1_v)

        def do_adds(b):
            def add_col(j, carry):
                for r in range(C_CH):
                    abufs[b][r, pl.ds(j * 16, 16)] = (
                        abufs[b][r, pl.ds(j * 16, 16)]
                        + bbufs[b][r, pl.ds(j * 16, 16)])
                return carry
            lax.fori_loop(0, DM // 16, add_col, 0)

        gha = [None] * nchunk
        ghb = [None] * nchunk
        wh = [None] * nchunk
        for i in range(nchunk):
            b = i % nb
            if i >= nb:
                wh[i - nb].wait()
            gha[i] = pltpu.async_copy(
                os_hbm.at[i0_v.at[pl.ds(i * C_CH, C_CH)]], abufs[b],
                gasems[b])
            ghb[i] = pltpu.async_copy(
                os_hbm.at[i1_v.at[pl.ds(i * C_CH, C_CH)]], bbufs[b],
                gbsems[b])
            if i >= 1:
                pb = (i - 1) % nb
                gha[i - 1].wait()
                ghb[i - 1].wait()
                do_adds(pb)
                wh[i - 1] = pltpu.async_copy(
                    abufs[pb], out_hbm.at[pl.ds(base + (i - 1) * C_CH, C_CH)],
                    wsems[pb])
        last = nchunk - 1
        gha[last].wait()
        ghb[last].wait()
        do_adds(last % nb)
        wh[last] = pltpu.async_copy(
            abufs[last % nb], out_hbm.at[pl.ds(base + last * C_CH, C_CH)],
            wsems[last % nb])
        for i in range(nchunk - nb, nchunk):
            wh[i].wait()

    return k(os, ts0, ts1)


def kernel(hidden_states, gate_w, e_score_correction_bias, w1, w2, w3):
    bsz, seq, dm = hidden_states.shape
    x = hidden_states.reshape(-1, dm)
    w_dense, w0r, w1r = _router(x, gate_w, e_score_correction_bias)
    ts0, ts1, ebx = _dispatch_plan(w_dense)
    nch = NTOK // SC_NW // C_D
    xs, ws_rep = _sc_dispatch(
        x, ts0.reshape(SC_NW, nch, C_D), ts1.reshape(SC_NW, nch, C_D),
        ts0, ts1, w0r, w1r)
    os = _ffn(xs, ws_rep, w1, w2, w3, ebx)
    final = _sc_combine(os, ts0, ts1)
    return final.reshape(bsz, seq, dm)


# speedup vs baseline: 1.0221x; 1.0003x over previous
import functools

import jax
import jax.numpy as jnp
from jax import lax
from jax.experimental import pallas as pl
from jax.experimental.pallas import tpu as pltpu
from jax.experimental.pallas import tpu_sc as plsc

NE = 8
DM = 2048
DFF = 1024
NTOK = 2048
BLK = 256
NSLOT = NTOK * 2 + NE * BLK
NBLK = NSLOT // BLK
TBLK = 256

SC_NC = 2
SC_NS = 16
SC_NW = SC_NC * SC_NS


def _router_body(x_ref, g_ref, b_ref, w_ref, wr0_ref, wr1_ref):
    x = x_ref[...]
    logits = lax.dot_general(x, g_ref[...], (((1,), (1,)), ((), ())),
                             preferred_element_type=jnp.float32)
    rw = jax.nn.sigmoid(logits)
    scores = rw + b_ref[...]
    ii = lax.broadcasted_iota(jnp.int32, scores.shape, 1)
    m1 = jnp.max(scores, axis=1, keepdims=True)
    i1 = jnp.min(jnp.where(scores == m1, ii, NE), axis=1, keepdims=True)
    s2 = jnp.where(ii == i1, -jnp.inf, scores)
    m2 = jnp.max(s2, axis=1, keepdims=True)
    i2 = jnp.min(jnp.where(s2 == m2, ii, NE), axis=1, keepdims=True)
    rw1 = jnp.sum(jnp.where(ii == i1, rw, 0.0), axis=1, keepdims=True)
    rw2 = jnp.sum(jnp.where(ii == i2, rw, 0.0), axis=1, keepdims=True)
    tot = rw1 + rw2
    w_ref[...] = (jnp.where(ii == i1, rw1 / tot, 0.0)
                  + jnp.where(ii == i2, rw2 / tot, 0.0))
    wlo = jnp.where(i1 < i2, rw1, rw2) / tot
    whi = jnp.where(i1 < i2, rw2, rw1) / tot
    wr0_ref[...] = lax.broadcast_in_dim(wlo, (TBLK, WREP), (0, 1))
    wr1_ref[...] = lax.broadcast_in_dim(whi, (TBLK, WREP), (0, 1))


def _router(x, gate_w, bias):
    return pl.pallas_call(
        _router_body,
        grid=(NTOK // TBLK,),
        in_specs=[
            pl.BlockSpec((TBLK, DM), lambda b: (b, 0)),
            pl.BlockSpec((NE, DM), lambda b: (0, 0)),
            pl.BlockSpec((1, NE), lambda b: (0, 0)),
        ],
        out_specs=[
            pl.BlockSpec((TBLK, NE), lambda b: (b, 0)),
            pl.BlockSpec((TBLK, WREP), lambda b: (b, 0)),
            pl.BlockSpec((TBLK, WREP), lambda b: (b, 0)),
        ],
        out_shape=[
            jax.ShapeDtypeStruct((NTOK, NE), jnp.float32),
            jax.ShapeDtypeStruct((NTOK, WREP), jnp.float32),
            jax.ShapeDtypeStruct((NTOK, WREP), jnp.float32),
        ],
    )(x, gate_w, bias.reshape(1, NE))


def _dispatch_plan(w_dense):
    mask = w_dense > 0.0
    mi = mask.astype(jnp.int32)
    counts = jnp.sum(mi, axis=0)
    padded = ((counts + BLK - 1) // BLK) * BLK
    pstart = jnp.concatenate(
        [jnp.zeros((1,), jnp.int32), jnp.cumsum(padded)[:-1].astype(jnp.int32)])
    prefix = jnp.cumsum(mi, axis=0) - mi
    slot = pstart[None, :] + prefix
    ts0 = jnp.min(jnp.where(mask, slot, NSLOT), axis=1).astype(jnp.int32)
    ts1 = jnp.max(jnp.where(mask, slot, -1), axis=1).astype(jnp.int32)
    bend = jnp.cumsum(padded // BLK)
    bidx = jnp.arange(NBLK, dtype=jnp.int32)
    eb = jnp.minimum(
        jnp.sum((bidx[:, None] >= bend[None, :]).astype(jnp.int32), axis=1),
        NE - 1).astype(jnp.int32)
    ebx = jnp.concatenate([eb, bend[-1:].astype(jnp.int32)])
    return ts0, ts1, ebx


C_D = 8
WREP = 128


def _sc_dispatch(x, t0_3d, t1_3d, ts0, ts1, w0r, w1r):
    tok_pw = NTOK // SC_NW
    nchunk = tok_pw // C_D
    nb = 3
    mesh = plsc.VectorSubcoreMesh(core_axis_name="c", subcore_axis_name="s")

    @functools.partial(
        pl.kernel, mesh=mesh,
        out_type=(jax.ShapeDtypeStruct((NSLOT, DM), jnp.float32),
                  jax.ShapeDtypeStruct((NSLOT, WREP), jnp.float32)),
        scratch_types=[
            pltpu.VMEM((nchunk, C_D), jnp.int32),
            pltpu.VMEM((nchunk, C_D), jnp.int32),
            pltpu.VMEM((tok_pw,), jnp.int32),
            pltpu.VMEM((tok_pw,), jnp.int32),
            pltpu.VMEM((tok_pw, WREP), jnp.float32),
            pltpu.VMEM((tok_pw, WREP), jnp.float32),
            pltpu.VMEM((C_D, DM), jnp.float32),
            pltpu.VMEM((C_D, DM), jnp.float32),
            pltpu.VMEM((C_D, DM), jnp.float32),
            pltpu.SemaphoreType.DMA,
            pltpu.SemaphoreType.DMA,
            pltpu.SemaphoreType.DMA,
            pltpu.SemaphoreType.DMA,
            pltpu.SemaphoreType.DMA,
            pltpu.SemaphoreType.DMA,
            pltpu.SemaphoreType.DMA,
            pltpu.SemaphoreType.DMA,
            pltpu.SemaphoreType.DMA,
            pltpu.SemaphoreType.DMA,
        ],
    )
    def k(x_hbm, t0_hbm, t1_hbm, tf0_hbm, tf1_hbm, w0_hbm, w1_hbm,
          xs_hbm, ws_hbm, i0_v, i1_v, if0_v, if1_v, w0_v, w1_v, b0, b1, b2,
          r0, r1, r2, s00, s01, s02, s10, s11, s12, wsem):
        bufs = (b0, b1, b2)
        rsems = (r0, r1, r2)
        s0sems = (s00, s01, s02)
        s1sems = (s10, s11, s12)
        wid = lax.axis_index("s") * SC_NC + lax.axis_index("c")
        base = wid * tok_pw
        pltpu.sync_copy(t0_hbm.at[wid], i0_v)
        pltpu.sync_copy(t1_hbm.at[wid], i1_v)
        pltpu.sync_copy(tf0_hbm.at[pl.ds(base, tok_pw)], if0_v)
        pltpu.sync_copy(tf1_hbm.at[pl.ds(base, tok_pw)], if1_v)
        pltpu.sync_copy(w0_hbm.at[pl.ds(base, tok_pw)], w0_v)
        pltpu.sync_copy(w1_hbm.at[pl.ds(base, tok_pw)], w1_v)
        wh0 = pltpu.async_copy(w0_v, ws_hbm.at[if0_v], wsem)
        wh1 = pltpu.async_copy(w1_v, ws_hbm.at[if1_v], wsem)
        rh = [None] * nchunk
        s0h = [None] * nchunk
        s1h = [None] * nchunk
        for i in range(nchunk):
            b = i % nb
            if i >= nb:
                s0h[i - nb].wait()
                s1h[i - nb].wait()
            rh[i] = pltpu.async_copy(
                x_hbm.at[pl.ds(base + i * C_D, C_D)], bufs[b], rsems[b])
            if i >= 1:
                pb = (i - 1) % nb
                rh[i - 1].wait()
                s0h[i - 1] = pltpu.async_copy(
                    bufs[pb], xs_hbm.at[i0_v.at[i - 1]], s0sems[pb])
                s1h[i - 1] = pltpu.async_copy(
                    bufs[pb], xs_hbm.at[i1_v.at[i - 1]], s1sems[pb])
        last = nchunk - 1
        rh[last].wait()
        s0h[last] = pltpu.async_copy(
            bufs[last % nb], xs_hbm.at[i0_v.at[last]], s0sems[last % nb])
        s1h[last] = pltpu.async_copy(
            bufs[last % nb], xs_hbm.at[i1_v.at[last]], s1sems[last % nb])
        for i in range(nchunk - nb, nchunk):
            s0h[i].wait()
            s1h[i].wait()
        wh0.wait()
        wh1.wait()

    return k(x, t0_3d, t1_3d, ts0, ts1, w0r, w1r)


def _ffn_body(eb_ref, xs_ref, ws_ref, w1_ref, w2_ref, w3_ref, os_ref):
    @pl.when(pl.program_id(0) < eb_ref[NBLK])
    def _compute():
        xb = xs_ref[...]
        a = lax.dot_general(xb, w1_ref[0], (((1,), (1,)), ((), ())),
                            preferred_element_type=jnp.float32)
        c = lax.dot_general(xb, w3_ref[0], (((1,), (1,)), ((), ())),
                            preferred_element_type=jnp.float32)
        h = a * jax.nn.sigmoid(a) * c
        o = lax.dot_general(h, w2_ref[0], (((1,), (1,)), ((), ())),
                            preferred_element_type=jnp.float32)
        os_ref[...] = o * ws_ref[:, :1]


def _ffn(xs, ws_rep, w1, w2, w3, eb):
    grid_spec = pltpu.PrefetchScalarGridSpec(
        num_scalar_prefetch=1,
        grid=(NBLK,),
        in_specs=[
            pl.BlockSpec((BLK, DM), lambda b, eb: (b, 0)),
            pl.BlockSpec((BLK, WREP), lambda b, eb: (b, 0)),
            pl.BlockSpec((1, DFF, DM), lambda b, eb: (eb[b], 0, 0)),
            pl.BlockSpec((1, DM, DFF), lambda b, eb: (eb[b], 0, 0)),
            pl.BlockSpec((1, DFF, DM), lambda b, eb: (eb[b], 0, 0)),
        ],
        out_specs=pl.BlockSpec((BLK, DM), lambda b, eb: (b, 0)),
    )
    return pl.pallas_call(
        _ffn_body,
        grid_spec=grid_spec,
        out_shape=jax.ShapeDtypeStruct((NSLOT, DM), jnp.float32),
    )(eb, xs, ws_rep, w1, w2, w3)


C_CH = 8


def _sc_combine(os, ts0, ts1):
    tok_pw = NTOK // SC_NW
    nchunk = tok_pw // C_CH
    nb = 3
    mesh = plsc.VectorSubcoreMesh(core_axis_name="c", subcore_axis_name="s")

    @functools.partial(
        pl.kernel, mesh=mesh,
        out_type=jax.ShapeDtypeStruct((NTOK, DM), jnp.float32),
        scratch_types=[
            pltpu.VMEM((tok_pw,), jnp.int32),
            pltpu.VMEM((tok_pw,), jnp.int32),
            pltpu.VMEM((C_CH, DM), jnp.float32),
            pltpu.VMEM((C_CH, DM), jnp.float32),
            pltpu.VMEM((C_CH, DM), jnp.float32),
            pltpu.VMEM((C_CH, DM), jnp.float32),
            pltpu.VMEM((C_CH, DM), jnp.float32),
            pltpu.VMEM((C_CH, DM), jnp.float32),
            pltpu.SemaphoreType.DMA,
            pltpu.SemaphoreType.DMA,
            pltpu.SemaphoreType.DMA,
            pltpu.SemaphoreType.DMA,
            pltpu.SemaphoreType.DMA,
            pltpu.SemaphoreType.DMA,
            pltpu.SemaphoreType.DMA,
            pltpu.SemaphoreType.DMA,
            pltpu.SemaphoreType.DMA,
        ],
    )
    def k(os_hbm, t0_hbm, t1_hbm, out_hbm, i0_v, i1_v, a0, a1, a2,
          c0, c1, c2, sa0, sa1, sa2, sb0, sb1, sb2, so0, so1, so2):
        abufs = (a0, a1, a2)
        bbufs = (c0, c1, c2)
        gasems = (sa0, sa1, sa2)
        gbsems = (sb0, sb1, sb2)
        wsems = (so0, so1, so2)
        wid = lax.axis_index("s") * SC_NC + lax.axis_index("c")
        base = wid * tok_pw
        pltpu.sync_copy(t0_hbm.at[pl.ds(base, tok_pw)], i0_v)
        pltpu.sync_copy(t1_hbm.at[pl.ds(base, tok_pw)], i1_v)

        def do_adds(b):
            def add_col(j, carry):
                for r in range(C_CH):
                    abufs[b][r, pl.ds(j * 16, 16)] = (
                        abufs[b][r, pl.ds(j * 16, 16)]
                        + bbufs[b][r, pl.ds(j * 16, 16)])
                return carry
            lax.fori_loop(0, DM // 16, add_col, 0)

        gha = [None] * nchunk
        ghb = [None] * nchunk
        wh = [None] * nchunk
        for i in range(nchunk):
            b = i % nb
            if i >= nb:
                wh[i - nb].wait()
            gha[i] = pltpu.async_copy(
                os_hbm.at[i0_v.at[pl.ds(i * C_CH, C_CH)]], abufs[b],
                gasems[b])
            ghb[i] = pltpu.async_copy(
                os_hbm.at[i1_v.at[pl.ds(i * C_CH, C_CH)]], bbufs[b],
                gbsems[b])
            if i >= 1:
                pb = (i - 1) % nb
                gha[i - 1].wait()
                ghb[i - 1].wait()
                do_adds(pb)
                wh[i - 1] = pltpu.async_copy(
                    abufs[pb], out_hbm.at[pl.ds(base + (i - 1) * C_CH, C_CH)],
                    wsems[pb])
        last = nchunk - 1
        gha[last].wait()
        ghb[last].wait()
        do_adds(last % nb)
        wh[last] = pltpu.async_copy(
            abufs[last % nb], out_hbm.at[pl.ds(base + last * C_CH, C_CH)],
            wsems[last % nb])
        for i in range(nchunk - nb, nchunk):
            wh[i].wait()

    return k(os, ts0, ts1)


def kernel(hidden_states, gate_w, e_score_correction_bias, w1, w2, w3):
    bsz, seq, dm = hidden_states.shape
    x = hidden_states.reshape(-1, dm)
    w_dense, w0r, w1r = _router(x, gate_w, e_score_correction_bias)
    ts0, ts1, ebx = _dispatch_plan(w_dense)
    nch = NTOK // SC_NW // C_D
    xs, ws_rep = _sc_dispatch(
        x, ts0.reshape(SC_NW, nch, C_D), ts1.reshape(SC_NW, nch, C_D),
        ts0, ts1, w0r, w1r)
    os = _ffn(xs, ws_rep, w1, w2, w3, ebx)
    final = _sc_combine(os, ts0, ts1)
    return final.reshape(bsz, seq, dm)
